# baseline (speedup 1.0000x reference)
import functools

import jax
import jax.numpy as jnp
from jax import lax
from jax.experimental import pallas as pl
from jax.experimental.pallas import tpu as pltpu
from jax.experimental.pallas import tpu_sc as plsc

N_NODES = 10000
E_EDGES = 320000
DIM = 128
EPS = 1e-5

NUM_CORES = 2
NUM_SUBCORES = 16
NW = NUM_CORES * NUM_SUBCORES
CHUNK = 80
EDGES_PER_W = E_EDGES // NW
CHUNKS_PER_W = EDGES_PER_W // CHUNK
N_REFILL = 5
CHUNKS_PER_REFILL = CHUNKS_PER_W // N_REFILL
TILE_ROWS = 640
N_PAD = TILE_ROWS * NUM_SUBCORES
SEGS = TILE_ROWS // CHUNK
ROW_BLK = 1000
N_BLKS = N_NODES // ROW_BLK

_MESH = plsc.VectorSubcoreMesh(core_axis_name="c", subcore_axis_name="s")


@functools.partial(
    pl.kernel,
    mesh=_MESH,
    out_type=jax.ShapeDtypeStruct((NUM_CORES, N_PAD, DIM), jnp.float32),
    scratch_types=[
        pltpu.VMEM((CHUNKS_PER_REFILL, CHUNK), jnp.int32),
        pltpu.VMEM((CHUNKS_PER_REFILL, CHUNK), jnp.int32),
        pltpu.VMEM((SEGS, CHUNK), jnp.int32),
        pltpu.VMEM((CHUNK, DIM), jnp.float32),
        pltpu.VMEM((CHUNK, DIM), jnp.float32),
        pltpu.VMEM((CHUNK, DIM), jnp.float32),
        pltpu.VMEM_SHARED((N_PAD, DIM), jnp.float32),
        pltpu.SemaphoreType.DMA,
        pltpu.SemaphoreType.DMA,
        pltpu.SemaphoreType.DMA,
        pltpu.SemaphoreType.DMA,
        pltpu.SemaphoreType.DMA,
        pltpu.SemaphoreType.DMA,
    ],
)
def _sc_aggregate(x_hbm, src_hbm, dst_hbm, zrows_hbm, iota_hbm,
                  psum_hbm, src_v, dst_v, iota_v, rows_a, rows_b, rows_c,
                  accum_sh, gsem_a, gsem_b, gsem_c, ssem_a, ssem_b, ssem_c):
    c = lax.axis_index("c")
    s = lax.axis_index("s")
    wid = s * NUM_CORES + c
    bufs = (rows_a, rows_b, rows_c)
    gsems = (gsem_a, gsem_b, gsem_c)
    ssems = (ssem_a, ssem_b, ssem_c)

    pltpu.sync_copy(iota_hbm.at[s], iota_v)
    pltpu.sync_copy(zrows_hbm, rows_a)
    for k in range(SEGS):
        pltpu.sync_copy(rows_a, accum_sh.at[iota_v.at[k]])
    plsc.subcore_barrier()

    def outer(g, carry):
        pltpu.sync_copy(src_hbm.at[wid, g], src_v)
        pltpu.sync_copy(dst_hbm.at[wid, g], dst_v)

        n = CHUNKS_PER_REFILL
        gdesc = [None] * n
        sdesc = [None] * n
        gdesc[0] = pltpu.async_copy(x_hbm.at[src_v.at[0]], bufs[0], gsems[0])
        gdesc[1] = pltpu.async_copy(x_hbm.at[src_v.at[1]], bufs[1], gsems[1])
        for i in range(n):
            if i >= 1:
                sdesc[i - 1].wait()
            if i + 2 < n:
                b = (i + 2) % 3
                gdesc[i + 2] = pltpu.async_copy(
                    x_hbm.at[src_v.at[i + 2]], bufs[b], gsems[b])
            gdesc[i].wait()
            sdesc[i] = pltpu.async_copy(
                bufs[i % 3], accum_sh.at[dst_v.at[i]], ssems[i % 3], add=True)
        sdesc[n - 1].wait()
        return carry

    lax.fori_loop(0, N_REFILL, outer, 0)
    plsc.subcore_barrier()

    for k in range(SEGS):
        pltpu.async_copy(accum_sh.at[iota_v.at[k]], rows_a, gsem_a).wait()
        obase = pl.multiple_of(s * TILE_ROWS + k * CHUNK, 8)
        pltpu.sync_copy(rows_a, psum_hbm.at[c, pl.ds(obase, CHUNK)])


@functools.partial(
    pl.kernel,
    mesh=_MESH,
    out_type=jax.ShapeDtypeStruct((NUM_CORES, N_PAD, DIM), jnp.float32),
    scratch_types=[
        pltpu.VMEM((CHUNKS_PER_REFILL, CHUNK), jnp.int32),
        pltpu.VMEM((SEGS, CHUNK), jnp.int32),
        pltpu.VMEM((CHUNK, DIM), jnp.float32),
        pltpu.VMEM((CHUNK, DIM), jnp.float32),
        pltpu.VMEM_SHARED((N_PAD, DIM), jnp.float32),
        pltpu.SemaphoreType.DMA,
    ],
)
def _sc_count(dst_hbm, zrows_hbm, orows_hbm, iota_hbm,
              pcnt_hbm, dst_v, iota_v, rows_v, ones_v, cnt_sh, sem):
    c = lax.axis_index("c")
    s = lax.axis_index("s")
    wid = s * NUM_CORES + c

    pltpu.sync_copy(iota_hbm.at[s], iota_v)
    pltpu.sync_copy(zrows_hbm, rows_v)
    pltpu.sync_copy(orows_hbm, ones_v)
    for k in range(SEGS):
        pltpu.sync_copy(rows_v, cnt_sh.at[iota_v.at[k]])
    plsc.subcore_barrier()

    def outer(g, carry):
        pltpu.sync_copy(dst_hbm.at[wid, g], dst_v)

        descs = [
            pltpu.async_copy(ones_v, cnt_sh.at[dst_v.at[j]], sem, add=True)
            for j in range(CHUNKS_PER_REFILL)
        ]
        for d in descs:
            d.wait()
        return carry

    lax.fori_loop(0, N_REFILL, outer, 0)
    plsc.subcore_barrier()

    for k in range(SEGS):
        pltpu.async_copy(cnt_sh.at[iota_v.at[k]], rows_v, sem).wait()
        obase = pl.multiple_of(s * TILE_ROWS + k * CHUNK, 8)
        pltpu.sync_copy(rows_v, pcnt_hbm.at[c, pl.ds(obase, CHUNK)])


def _linear_body(p0_ref, p1_ref, c0_ref, c1_ref, x_ref, wl_ref, wr_ref,
                 bl_ref, h_ref, stats_ref):
    i = pl.program_id(0)
    cnt = c0_ref[0][:, 0:1] + c1_ref[0][:, 0:1]
    cnt = jnp.maximum(cnt, 1.0)
    aggr = (p0_ref[0] + p1_ref[0]) / cnt
    h = lax.dot_general(aggr, wl_ref[...], (((1,), (1,)), ((), ())),
                        preferred_element_type=jnp.float32)
    h = h + lax.dot_general(x_ref[...], wr_ref[...], (((1,), (1,)), ((), ())),
                            preferred_element_type=jnp.float32)
    h = h + bl_ref[...]
    h_ref[...] = h

    @pl.when(i == 0)
    def _init():
        stats_ref[0] = 0.0
        stats_ref[1] = 0.0

    stats_ref[0] = stats_ref[0] + jnp.sum(h)
    stats_ref[1] = stats_ref[1] + jnp.sum(h * h)


_tc_linear = pl.pallas_call(
    _linear_body,
    grid=(N_BLKS,),
    in_specs=[
        pl.BlockSpec((1, ROW_BLK, DIM), lambda i: (0, i, 0)),
        pl.BlockSpec((1, ROW_BLK, DIM), lambda i: (1, i, 0)),
        pl.BlockSpec((1, ROW_BLK, DIM), lambda i: (0, i, 0)),
        pl.BlockSpec((1, ROW_BLK, DIM), lambda i: (1, i, 0)),
        pl.BlockSpec((ROW_BLK, DIM), lambda i: (i, 0)),
        pl.BlockSpec((DIM, DIM), lambda i: (0, 0)),
        pl.BlockSpec((DIM, DIM), lambda i: (0, 0)),
        pl.BlockSpec((1, DIM), lambda i: (0, 0)),
    ],
    out_specs=[
        pl.BlockSpec((ROW_BLK, DIM), lambda i: (i, 0)),
        pl.BlockSpec(memory_space=pltpu.SMEM),
    ],
    out_shape=[
        jax.ShapeDtypeStruct((N_NODES, DIM), jnp.float32),
        jax.ShapeDtypeStruct((2,), jnp.float32),
    ],
)


def _norm_body(h_ref, stats_ref, lnw_ref, lnb_ref, o_ref):
    total = float(N_NODES * DIM)
    mean = stats_ref[0] / total
    var = stats_ref[1] / total - mean * mean
    var = jnp.maximum(var, 0.0)
    inv = 1.0 / (jnp.sqrt(var) + EPS)
    o_ref[...] = (h_ref[...] - mean) * (inv * lnw_ref[...]) + lnb_ref[...]


_tc_norm = pl.pallas_call(
    _norm_body,
    grid=(N_BLKS,),
    in_specs=[
        pl.BlockSpec((ROW_BLK, DIM), lambda i: (i, 0)),
        pl.BlockSpec(memory_space=pltpu.SMEM),
        pl.BlockSpec((1, DIM), lambda i: (0, 0)),
        pl.BlockSpec((1, DIM), lambda i: (0, 0)),
    ],
    out_specs=pl.BlockSpec((ROW_BLK, DIM), lambda i: (i, 0)),
    out_shape=jax.ShapeDtypeStruct((N_NODES, DIM), jnp.float32),
)


def kernel(x, edge_index, W_l, b_l, W_r, ln_w, ln_b):
    src4d = edge_index[0].reshape(NW, N_REFILL, CHUNKS_PER_REFILL, CHUNK)
    dst4d = edge_index[1].reshape(NW, N_REFILL, CHUNKS_PER_REFILL, CHUNK)
    zrows = jnp.zeros((CHUNK, DIM), jnp.float32)
    orows = jnp.ones((CHUNK, DIM), jnp.float32)
    iota3d = jnp.arange(N_PAD, dtype=jnp.int32).reshape(
        NUM_SUBCORES, SEGS, CHUNK)
    psum = _sc_aggregate(x, src4d, dst4d, zrows, iota3d)
    pcnt = _sc_count(dst4d, zrows, orows, iota3d)
    h, stats = _tc_linear(psum, psum, pcnt, pcnt, x, W_l, W_r,
                          b_l.reshape(1, DIM))
    return _tc_norm(h, stats, ln_w.reshape(1, DIM), ln_b.reshape(1, DIM))


# speedup vs baseline: 1.0181x; 1.0181x over previous
import functools

import jax
import jax.numpy as jnp
from jax import lax
from jax.experimental import pallas as pl
from jax.experimental.pallas import tpu as pltpu
from jax.experimental.pallas import tpu_sc as plsc

N_NODES = 10000
E_EDGES = 320000
DIM = 128
EPS = 1e-5

NUM_CORES = 2
NUM_SUBCORES = 16
NW = NUM_CORES * NUM_SUBCORES
CHUNK = 80
EDGES_PER_W = E_EDGES // NW
CHUNKS_PER_W = EDGES_PER_W // CHUNK
N_REFILL = 5
CHUNKS_PER_REFILL = CHUNKS_PER_W // N_REFILL
TILE_ROWS = 640
N_PAD = TILE_ROWS * NUM_SUBCORES
SEGS = TILE_ROWS // CHUNK
ROW_BLK = 1000
N_BLKS = N_NODES // ROW_BLK

_MESH = plsc.VectorSubcoreMesh(core_axis_name="c", subcore_axis_name="s")


@functools.partial(
    pl.kernel,
    mesh=_MESH,
    out_type=[
        jax.ShapeDtypeStruct((NUM_CORES, N_PAD, DIM), jnp.float32),
        jax.ShapeDtypeStruct((NUM_CORES, N_PAD, DIM), jnp.float32),
    ],
    scratch_types=[
        pltpu.VMEM((CHUNKS_PER_REFILL, CHUNK), jnp.int32),
        pltpu.VMEM((CHUNKS_PER_REFILL, CHUNK), jnp.int32),
        pltpu.VMEM((SEGS, CHUNK), jnp.int32),
        pltpu.VMEM((CHUNK, DIM), jnp.float32),
        pltpu.VMEM((CHUNK, DIM), jnp.float32),
        pltpu.VMEM((CHUNK, DIM), jnp.float32),
        pltpu.VMEM_SHARED((N_PAD, DIM), jnp.float32),
        pltpu.SemaphoreType.DMA,
        pltpu.SemaphoreType.DMA,
        pltpu.SemaphoreType.DMA,
        pltpu.SemaphoreType.DMA,
        pltpu.SemaphoreType.DMA,
        pltpu.SemaphoreType.DMA,
    ],
)
def _sc_aggregate(x_hbm, src_hbm, dst_hbm, zrows_hbm, orows_hbm, iota_hbm,
                  psum_hbm, pcnt_hbm,
                  src_v, dst_v, iota_v, rows_a, rows_b, rows_c,
                  accum_sh, gsem_a, gsem_b, gsem_c, ssem_a, ssem_b, ssem_c):
    c = lax.axis_index("c")
    s = lax.axis_index("s")
    wid = s * NUM_CORES + c
    bufs = (rows_a, rows_b, rows_c)
    gsems = (gsem_a, gsem_b, gsem_c)
    ssems = (ssem_a, ssem_b, ssem_c)

    pltpu.sync_copy(iota_hbm.at[s], iota_v)
    pltpu.sync_copy(zrows_hbm, rows_a)
    for k in range(SEGS):
        pltpu.sync_copy(rows_a, accum_sh.at[iota_v.at[k]])
    plsc.subcore_barrier()

    def outer(g, carry):
        pltpu.sync_copy(src_hbm.at[wid, g], src_v)
        pltpu.sync_copy(dst_hbm.at[wid, g], dst_v)

        n = CHUNKS_PER_REFILL
        gdesc = [None] * n
        sdesc = [None] * n
        gdesc[0] = pltpu.async_copy(x_hbm.at[src_v.at[0]], bufs[0], gsems[0])
        gdesc[1] = pltpu.async_copy(x_hbm.at[src_v.at[1]], bufs[1], gsems[1])
        for i in range(n):
            if i >= 1:
                sdesc[i - 1].wait()
            if i + 2 < n:
                b = (i + 2) % 3
                gdesc[i + 2] = pltpu.async_copy(
                    x_hbm.at[src_v.at[i + 2]], bufs[b], gsems[b])
            gdesc[i].wait()
            sdesc[i] = pltpu.async_copy(
                bufs[i % 3], accum_sh.at[dst_v.at[i]], ssems[i % 3], add=True)
        sdesc[n - 1].wait()
        return carry

    lax.fori_loop(0, N_REFILL, outer, 0)
    plsc.subcore_barrier()

    for k in range(SEGS):
        pltpu.async_copy(accum_sh.at[iota_v.at[k]], rows_a, gsem_a).wait()
        obase = pl.multiple_of(s * TILE_ROWS + k * CHUNK, 8)
        pltpu.sync_copy(rows_a, psum_hbm.at[c, pl.ds(obase, CHUNK)])

    pltpu.sync_copy(zrows_hbm, rows_a)
    pltpu.sync_copy(orows_hbm, rows_b)
    for k in range(SEGS):
        pltpu.sync_copy(rows_a, accum_sh.at[iota_v.at[k]])
    plsc.subcore_barrier()

    def outer_cnt(g, carry):
        pltpu.sync_copy(dst_hbm.at[wid, g], dst_v)
        descs = [
            pltpu.async_copy(rows_b, accum_sh.at[dst_v.at[j]], ssem_a,
                             add=True)
            for j in range(CHUNKS_PER_REFILL)
        ]
        for d in descs:
            d.wait()
        return carry

    lax.fori_loop(0, N_REFILL, outer_cnt, 0)
    plsc.subcore_barrier()

    for k in range(SEGS):
        pltpu.async_copy(accum_sh.at[iota_v.at[k]], rows_c, gsem_c).wait()
        obase = pl.multiple_of(s * TILE_ROWS + k * CHUNK, 8)
        pltpu.sync_copy(rows_c, pcnt_hbm.at[c, pl.ds(obase, CHUNK)])


def _linear_body(p0_ref, p1_ref, c0_ref, c1_ref, x_ref, wl_ref, wr_ref,
                 bl_ref, h_ref, stats_ref):
    i = pl.program_id(0)
    cnt = c0_ref[0][:, 0:1] + c1_ref[0][:, 0:1]
    cnt = jnp.maximum(cnt, 1.0)
    aggr = (p0_ref[0] + p1_ref[0]) / cnt
    h = lax.dot_general(aggr, wl_ref[...], (((1,), (1,)), ((), ())),
                        preferred_element_type=jnp.float32)
    h = h + lax.dot_general(x_ref[...], wr_ref[...], (((1,), (1,)), ((), ())),
                            preferred_element_type=jnp.float32)
    h = h + bl_ref[...]
    h_ref[...] = h

    @pl.when(i == 0)
    def _init():
        stats_ref[0] = 0.0
        stats_ref[1] = 0.0

    stats_ref[0] = stats_ref[0] + jnp.sum(h)
    stats_ref[1] = stats_ref[1] + jnp.sum(h * h)


_tc_linear = pl.pallas_call(
    _linear_body,
    grid=(N_BLKS,),
    in_specs=[
        pl.BlockSpec((1, ROW_BLK, DIM), lambda i: (0, i, 0)),
        pl.BlockSpec((1, ROW_BLK, DIM), lambda i: (1, i, 0)),
        pl.BlockSpec((1, ROW_BLK, DIM), lambda i: (0, i, 0)),
        pl.BlockSpec((1, ROW_BLK, DIM), lambda i: (1, i, 0)),
        pl.BlockSpec((ROW_BLK, DIM), lambda i: (i, 0)),
        pl.BlockSpec((DIM, DIM), lambda i: (0, 0)),
        pl.BlockSpec((DIM, DIM), lambda i: (0, 0)),
        pl.BlockSpec((1, DIM), lambda i: (0, 0)),
    ],
    out_specs=[
        pl.BlockSpec((ROW_BLK, DIM), lambda i: (i, 0)),
        pl.BlockSpec(memory_space=pltpu.SMEM),
    ],
    out_shape=[
        jax.ShapeDtypeStruct((N_NODES, DIM), jnp.float32),
        jax.ShapeDtypeStruct((2,), jnp.float32),
    ],
)


def _norm_body(h_ref, stats_ref, lnw_ref, lnb_ref, o_ref):
    total = float(N_NODES * DIM)
    mean = stats_ref[0] / total
    var = stats_ref[1] / total - mean * mean
    var = jnp.maximum(var, 0.0)
    inv = 1.0 / (jnp.sqrt(var) + EPS)
    o_ref[...] = (h_ref[...] - mean) * (inv * lnw_ref[...]) + lnb_ref[...]


_tc_norm = pl.pallas_call(
    _norm_body,
    grid=(N_BLKS,),
    in_specs=[
        pl.BlockSpec((ROW_BLK, DIM), lambda i: (i, 0)),
        pl.BlockSpec(memory_space=pltpu.SMEM),
        pl.BlockSpec((1, DIM), lambda i: (0, 0)),
        pl.BlockSpec((1, DIM), lambda i: (0, 0)),
    ],
    out_specs=pl.BlockSpec((ROW_BLK, DIM), lambda i: (i, 0)),
    out_shape=jax.ShapeDtypeStruct((N_NODES, DIM), jnp.float32),
)


def kernel(x, edge_index, W_l, b_l, W_r, ln_w, ln_b):
    src4d = edge_index[0].reshape(NW, N_REFILL, CHUNKS_PER_REFILL, CHUNK)
    dst4d = edge_index[1].reshape(NW, N_REFILL, CHUNKS_PER_REFILL, CHUNK)
    zrows = jnp.zeros((CHUNK, DIM), jnp.float32)
    orows = jnp.ones((CHUNK, DIM), jnp.float32)
    iota3d = jnp.arange(N_PAD, dtype=jnp.int32).reshape(
        NUM_SUBCORES, SEGS, CHUNK)
    psum, pcnt = _sc_aggregate(x, src4d, dst4d, zrows, orows, iota3d)
    h, stats = _tc_linear(psum, psum, pcnt, pcnt, x, W_l, W_r,
                          b_l.reshape(1, DIM))
    return _tc_norm(h, stats, ln_w.reshape(1, DIM), ln_b.reshape(1, DIM))
